# baseline (speedup 1.0000x reference)
import math

import jax
import jax.numpy as jnp
from jax.experimental import pallas as pl
from jax.experimental.pallas import tpu as pltpu

_QLEN = 4
_L = 128
_B = 16
_NHID = 512
_DK = 256
_N = 10
_K = 4
_DKB = 64
_NCHUNK = _DK // _DKB
_ROWS = _QLEN * _B
_SCALE = 1.0 / math.sqrt(_DK)
_NEG = -3.0e38


def _body(q_ref, w_ref, k_ref, b_ref, wout_ref, iout_ref, sc_ref):
    i = pl.program_id(0)
    acc0 = jnp.zeros((_ROWS, _DKB), jnp.float32)

    def step(l, acc):
        qj = q_ref[:, l].reshape(_ROWS, _NHID)
        wj = w_ref[:, pl.ds(l * _NHID, _NHID)]
        return acc + jax.lax.dot_general(
            qj, wj, (((1,), (1,)), ((), ())),
            preferred_element_type=jnp.float32)

    acc = jax.lax.fori_loop(0, _L, step, acc0, unroll=16)

    @pl.when(i == 0)
    def _init():
        sc_ref[...] = jnp.zeros_like(sc_ref)

    qdc = (acc + b_ref[:, i]).reshape(_QLEN, _B, _DKB)
    cols = []
    for n in range(_N):
        kn = k_ref[n, :, i]
        cols.append(jnp.sum(qdc * kn[None], axis=-1).reshape(_ROWS, 1))
    sc_ref[...] += jnp.concatenate(cols, axis=1) * _SCALE

    @pl.when(i == _NCHUNK - 1)
    def _epilogue():
        scores = sc_ref[...]
        m = jnp.max(scores, axis=-1, keepdims=True)
        e = jnp.exp(scores - m)
        p = e / jnp.sum(e, axis=-1, keepdims=True)
        iota = jax.lax.broadcasted_iota(jnp.int32, (_ROWS, _N), 1)
        work = p
        vals = []
        for j in range(_K):
            mv = jnp.max(work, axis=-1, keepdims=True)
            sel = work == mv
            idx = jnp.min(jnp.where(sel, iota, _N), axis=-1)
            vals.append(mv)
            iout_ref[:, j:j + 1] = idx.astype(jnp.int32).reshape(_ROWS, 1)
            work = jnp.where(iota == idx[:, None], _NEG, work)
        w4 = jnp.concatenate(vals, axis=1)
        m2 = jnp.max(w4, axis=-1, keepdims=True)
        e2 = jnp.exp(w4 - m2)
        wout_ref[...] = e2 / jnp.sum(e2, axis=-1, keepdims=True)


def kernel(query, keys, values, W, b):
    del values
    wk, ik = pl.pallas_call(
        _body,
        grid=(_NCHUNK,),
        in_specs=[
            pl.BlockSpec((_QLEN, _L, _B, _NHID), lambda i: (0, 0, 0, 0)),
            pl.BlockSpec((_DKB, _L * _NHID), lambda i: (i, 0)),
            pl.BlockSpec((_N, _B, _NCHUNK, _DKB), lambda i: (0, 0, 0, 0)),
            pl.BlockSpec((1, _NCHUNK, _DKB), lambda i: (0, 0, 0)),
        ],
        out_specs=[
            pl.BlockSpec((_ROWS, _K), lambda i: (0, 0)),
            pl.BlockSpec((_ROWS, _K), lambda i: (0, 0)),
        ],
        out_shape=[
            jax.ShapeDtypeStruct((_ROWS, _K), jnp.float32),
            jax.ShapeDtypeStruct((_ROWS, _K), jnp.int32),
        ],
        scratch_shapes=[pltpu.VMEM((_ROWS, _N), jnp.float32)],
        compiler_params=pltpu.CompilerParams(
            dimension_semantics=("arbitrary",),
        ),
    )(query, W, keys.reshape(_N, _B, _NCHUNK, _DKB),
      b.reshape(1, _NCHUNK, _DKB))
    return wk.reshape(_ROWS, 1, _K), ik.T


# speedup vs baseline: 1.0889x; 1.0889x over previous
import math

import jax
import jax.numpy as jnp
from jax.experimental import pallas as pl
from jax.experimental.pallas import tpu as pltpu

_QLEN = 4
_L = 128
_B = 16
_NHID = 512
_DK = 256
_N = 10
_K = 4
_DKB = 64
_NCHUNK = _DK // _DKB
_ROWS = _QLEN * _B
_SCALE = 1.0 / math.sqrt(_DK)
_NEG = -3.0e38


def _body(q_ref, w_ref, k_ref, b_ref, wout_ref, iout_ref, sum_ref):
    i = pl.program_id(0)
    acc0 = jnp.zeros((_ROWS, _DKB), jnp.float32)

    def step(l, acc):
        qj = q_ref[:, l].reshape(_ROWS, _NHID).astype(jnp.bfloat16)
        wj = w_ref[:, pl.ds(l * _NHID, _NHID)].astype(jnp.bfloat16)
        return acc + jax.lax.dot_general(
            qj, wj, (((1,), (1,)), ((), ())),
            preferred_element_type=jnp.float32)

    acc = jax.lax.fori_loop(0, _L, step, acc0, unroll=16)
    sum_ref[i] = acc

    @pl.when(i == _NCHUNK - 1)
    def _epilogue():
        qd = jnp.concatenate([sum_ref[c] for c in range(_NCHUNK)],
                             axis=1) + b_ref[...]
        qd3 = qd.reshape(_QLEN, _B, _DK)
        cols = []
        qd3 = qd3.astype(jnp.bfloat16).astype(jnp.float32)
        for n in range(_N):
            kn = k_ref[n].astype(jnp.bfloat16).astype(jnp.float32)
            cols.append(jnp.sum(qd3 * kn[None], axis=-1).reshape(_ROWS, 1))
        scores = jnp.concatenate(cols, axis=1) * _SCALE
        m = jnp.max(scores, axis=-1, keepdims=True)
        e = jnp.exp(scores - m)
        p = e / jnp.sum(e, axis=-1, keepdims=True)
        iota = jax.lax.broadcasted_iota(jnp.int32, (_ROWS, _N), 1)
        work = p
        vals = []
        for j in range(_K):
            mv = jnp.max(work, axis=-1, keepdims=True)
            sel = work == mv
            idx = jnp.min(jnp.where(sel, iota, _N), axis=-1)
            vals.append(mv)
            iout_ref[:, j:j + 1] = idx.astype(jnp.int32).reshape(_ROWS, 1)
            work = jnp.where(iota == idx[:, None], _NEG, work)
        w4 = jnp.concatenate(vals, axis=1)
        m2 = jnp.max(w4, axis=-1, keepdims=True)
        e2 = jnp.exp(w4 - m2)
        wout_ref[...] = e2 / jnp.sum(e2, axis=-1, keepdims=True)


def kernel(query, keys, values, W, b):
    del values
    b2 = b.reshape(1, _DK)
    wk, ik = pl.pallas_call(
        _body,
        grid=(_NCHUNK,),
        in_specs=[
            pl.BlockSpec((_QLEN, _L, _B, _NHID), lambda i: (0, 0, 0, 0)),
            pl.BlockSpec((_DKB, _L * _NHID), lambda i: (i, 0)),
            pl.BlockSpec((_N, _B, _DK), lambda i: (0, 0, 0)),
            pl.BlockSpec((1, _DK), lambda i: (0, 0)),
        ],
        out_specs=[
            pl.BlockSpec((_ROWS, _K), lambda i: (0, 0)),
            pl.BlockSpec((_ROWS, _K), lambda i: (0, 0)),
        ],
        out_shape=[
            jax.ShapeDtypeStruct((_ROWS, _K), jnp.float32),
            jax.ShapeDtypeStruct((_ROWS, _K), jnp.int32),
        ],
        scratch_shapes=[pltpu.VMEM((_NCHUNK, _ROWS, _DKB), jnp.float32)],
        compiler_params=pltpu.CompilerParams(
            dimension_semantics=("arbitrary",),
        ),
    )(query, W, keys, b2)
    return wk.reshape(_ROWS, 1, _K), ik.T


# speedup vs baseline: 1.1167x; 1.0256x over previous
import math

import jax
import jax.numpy as jnp
from jax.experimental import pallas as pl
from jax.experimental.pallas import tpu as pltpu

_QLEN = 4
_L = 128
_B = 16
_NHID = 512
_DK = 256
_N = 10
_K = 4
_DKB = 64
_NCHUNK = _DK // _DKB
_LSPLIT = 2
_ROWS = _QLEN * _B
_SCALE = 1.0 / math.sqrt(_DK)
_NEG = -3.0e38


def _body(q_ref, w_ref, k_ref, b_ref, wout_ref, iout_ref, sum_ref):
    i = pl.program_id(0)
    j = pl.program_id(1)
    lbase = j * (_L // _LSPLIT)

    def step(l, acc):
        qj = q_ref[:, lbase + l].reshape(_ROWS, _NHID).astype(jnp.bfloat16)
        wj = w_ref[:, pl.ds(l * _NHID, _NHID)].astype(jnp.bfloat16)
        return acc + jax.lax.dot_general(
            qj, wj, (((1,), (1,)), ((), ())),
            preferred_element_type=jnp.float32)

    @pl.when(j == 0)
    def _initacc():
        sum_ref[i] = jnp.zeros((_ROWS, _DKB), jnp.float32)

    acc0 = sum_ref[i]
    sum_ref[i] = jax.lax.fori_loop(0, _L // _LSPLIT, step, acc0, unroll=16)

    @pl.when((i == _NCHUNK - 1) & (j == _LSPLIT - 1))
    def _epilogue():
        qd = jnp.concatenate([sum_ref[c] for c in range(_NCHUNK)],
                             axis=1) + b_ref[...]
        qd3 = qd.reshape(_QLEN, _B, _DK)
        cols = []
        qd3 = qd3.astype(jnp.bfloat16).astype(jnp.float32)
        for n in range(_N):
            kn = k_ref[n].astype(jnp.bfloat16).astype(jnp.float32)
            cols.append(jnp.sum(qd3 * kn[None], axis=-1).reshape(_ROWS, 1))
        scores = jnp.concatenate(cols, axis=1) * _SCALE
        m = jnp.max(scores, axis=-1, keepdims=True)
        e = jnp.exp(scores - m)
        p = e / jnp.sum(e, axis=-1, keepdims=True)
        iota = jax.lax.broadcasted_iota(jnp.int32, (_ROWS, _N), 1)
        work = p
        vals = []
        for j in range(_K):
            mv = jnp.max(work, axis=-1, keepdims=True)
            sel = work == mv
            idx = jnp.min(jnp.where(sel, iota, _N), axis=-1)
            vals.append(mv)
            iout_ref[:, j:j + 1] = idx.astype(jnp.int32).reshape(_ROWS, 1)
            work = jnp.where(iota == idx[:, None], _NEG, work)
        w4 = jnp.concatenate(vals, axis=1)
        m2 = jnp.max(w4, axis=-1, keepdims=True)
        e2 = jnp.exp(w4 - m2)
        wout_ref[...] = e2 / jnp.sum(e2, axis=-1, keepdims=True)


def kernel(query, keys, values, W, b):
    del values
    b2 = b.reshape(1, _DK)
    wk, ik = pl.pallas_call(
        _body,
        grid=(_NCHUNK, _LSPLIT),
        in_specs=[
            pl.BlockSpec((_QLEN, _L, _B, _NHID), lambda i, j: (0, 0, 0, 0)),
            pl.BlockSpec((_DKB, _L * _NHID // _LSPLIT), lambda i, j: (i, j)),
            pl.BlockSpec((_N, _B, _DK), lambda i, j: (0, 0, 0)),
            pl.BlockSpec((1, _DK), lambda i, j: (0, 0)),
        ],
        out_specs=[
            pl.BlockSpec((_ROWS, _K), lambda i, j: (0, 0)),
            pl.BlockSpec((_ROWS, _K), lambda i, j: (0, 0)),
        ],
        out_shape=[
            jax.ShapeDtypeStruct((_ROWS, _K), jnp.float32),
            jax.ShapeDtypeStruct((_ROWS, _K), jnp.int32),
        ],
        scratch_shapes=[pltpu.VMEM((_NCHUNK, _ROWS, _DKB), jnp.float32)],
        compiler_params=pltpu.CompilerParams(
            dimension_semantics=("arbitrary", "arbitrary"),
        ),
    )(query, W, keys, b2)
    return wk.reshape(_ROWS, 1, _K), ik.T
